# Initial kernel scaffold; baseline (speedup 1.0000x reference)
import jax
import jax.numpy as jnp
from jax.experimental import pallas as pl


def kernel(x):
    raise NotImplementedError("write your pallas kernel here")


# speedup vs baseline: 3.1703x; 3.1703x over previous
import jax
import jax.numpy as jnp
from jax.experimental import pallas as pl
from jax.experimental.pallas import tpu as pltpu

R = 7
H = 2 ** R
W = 2 ** (R + 1)
NB = 8
NC = 32
CH = 5
HB = H // 2


def _masked_sum(blk, masks):
    m = masks[0]
    for extra in masks[1:]:
        m = m | extra
    return jnp.sum(jnp.where(m, blk, 0.0))


def _body(x_ref, o_ref, sm_ref):
    j = pl.program_id(1)
    o_ref[...] = x_ref[...]

    @pl.when(j == 0)
    def _bottom():
        b7 = x_ref[0, :, :, HB - 1:HB, :]
        ci = jax.lax.broadcasted_iota(jnp.int32, (NC, CH, 1, W), 1)
        wi = jax.lax.broadcasted_iota(jnp.int32, (NC, CH, 1, W), 3)
        for c in range(CH):
            src = (c - 1) % CH
            sm_ref[0, c] = _masked_sum(
                b7, [(ci == src) & ((wi == H - 1) | (wi == H))])
            sm_ref[1, c] = _masked_sum(b7, [(ci == src) & (wi == W - 1)])

    @pl.when(j == 1)
    def _top():
        t01 = x_ref[0, :, :, 0:2, :]
        ci = jax.lax.broadcasted_iota(jnp.int32, (NC, CH, 2, W), 1)
        hi = jax.lax.broadcasted_iota(jnp.int32, (NC, CH, 2, W), 2)
        wi = jax.lax.broadcasted_iota(jnp.int32, (NC, CH, 2, W), 3)
        row = x_ref[0, :, :, 0:1, :]
        ci_r = jax.lax.broadcasted_iota(jnp.int32, (NC, CH, 1, W), 1)
        wi_r = jax.lax.broadcasted_iota(jnp.int32, (NC, CH, 1, W), 3)
        acc = row
        for c in range(CH):
            s1 = _masked_sum(t01, [
                (ci == c) & (hi == 1) & ((wi == 0) | (wi == 1)),
                (ci == c) & (hi == 0) & (wi == 1),
            ])
            s2 = _masked_sum(t01, [
                (ci == c) & (hi == 1) & ((wi == H) | (wi == H + 1)),
                (ci == c) & (hi == 0) & ((wi == H + 1) | (wi == H - 1)),
            ])
            m1 = (sm_ref[0, c] + s1) * (1.0 / 160.0)
            m2 = (sm_ref[1, c] + s2) * (1.0 / 160.0)
            acc = jnp.where((ci_r == c) & (wi_r == 0), m1, acc)
            acc = jnp.where((ci_r == c) & (wi_r == H), m2, acc)
        o_ref[0, :, :, 0:1, :] = acc


def kernel(x):
    return pl.pallas_call(
        _body,
        grid=(NB, 2),
        in_specs=[pl.BlockSpec((1, NC, CH, HB, W),
                               lambda b, j: (b, 0, 0, 1 - j, 0))],
        out_specs=pl.BlockSpec((1, NC, CH, HB, W),
                               lambda b, j: (b, 0, 0, 1 - j, 0)),
        out_shape=jax.ShapeDtypeStruct((NB, NC, CH, H, W), jnp.float32),
        scratch_shapes=[pltpu.SMEM((2, CH), jnp.float32)],
    )(x)
